# baseline (speedup 1.0000x reference)
import jax
import jax.numpy as jnp
from jax import lax
from jax.experimental import pallas as pl
from jax.experimental.pallas import tpu as pltpu
from jax.experimental.pallas import tpu_sc as plsc
from jax._src.pallas import mpmd as _plmpmd

R, C = 128, 32768
BC = 4096
NB = C // BC
BZ = R * C // NB
INT_MAX = 2147483647

NC, NS = 2, 16
NW = NC * NS
ROWS_PER_W = R // NW
REP = 16 // ROWS_PER_W


def _amax_zero_body(x_ref, zero_ref, idxrep_ref, m_scr, i_scr):
    j = pl.program_id(0)
    blk = x_ref[...]
    m = jnp.max(blk, axis=1, keepdims=True)
    liota = lax.broadcasted_iota(jnp.int32, blk.shape, 1) + j * BC
    cand = jnp.where(blk == m, liota, INT_MAX)
    ci = jnp.min(cand, axis=1, keepdims=True)

    @pl.when(j == 0)
    def _():
        m_scr[...] = m
        i_scr[...] = ci

    @pl.when(j > 0)
    def _():
        upd = m > m_scr[...]
        i_scr[...] = jnp.where(upd, ci, i_scr[...])
        m_scr[...] = jnp.where(upd, m, m_scr[...])

    zero_ref[...] = jnp.zeros((R, BC), jnp.float32)

    @pl.when(j == NB - 1)
    def _():
        row = lax.broadcasted_iota(jnp.int32, (R, REP), 0)
        idxrep_ref[...] = i_scr[...] + row * C


def _scatter_body(idx_hbm, zeros_hbm, out_hbm, idx_v, ones_v, sem):
    del zeros_hbm
    w = lax.axis_index("s") * NC + lax.axis_index("c")
    ones_v[...] = jnp.ones((16,), jnp.float32)
    pltpu.sync_copy(idx_hbm.at[pl.ds(w * 16, 16)], idx_v)
    pltpu.async_copy(ones_v, out_hbm.at[idx_v], sem).wait()


_sc_scatter = _plmpmd._mpmd_map(
    [(
        plsc.VectorSubcoreMesh(core_axis_name="c", subcore_axis_name="s"),
        _scatter_body,
    )],
    out_types=jax.ShapeDtypeStruct((R * C,), jnp.float32),
    input_output_aliases={1: 0},
    scratch_types=[
        pltpu.VMEM((16,), jnp.int32),
        pltpu.VMEM((16,), jnp.float32),
        pltpu.SemaphoreType.DMA,
    ],
)


def kernel(x):
    zeros2d, idxrep = pl.pallas_call(
        _amax_zero_body,
        grid=(NB,),
        in_specs=[pl.BlockSpec((R, BC), lambda j: (0, j))],
        out_specs=[
            pl.BlockSpec((R, BC), lambda j: (0, j)),
            pl.BlockSpec((R, REP), lambda j: (0, 0)),
        ],
        out_shape=[
            jax.ShapeDtypeStruct((R, C), jnp.float32),
            jax.ShapeDtypeStruct((R, REP), jnp.int32),
        ],
        scratch_shapes=[
            pltpu.VMEM((R, 1), jnp.float32),
            pltpu.VMEM((R, 1), jnp.int32),
        ],
    )(x)

    idx_flat = idxrep.reshape(R * REP)
    out_flat = _sc_scatter(idx_flat, zeros2d.reshape(R * C))
    return out_flat.reshape(R, C)


# speedup vs baseline: 2.1989x; 2.1989x over previous
import jax
import jax.numpy as jnp
from jax import lax
from jax.experimental import pallas as pl
from jax.experimental.pallas import tpu as pltpu

R, C = 128, 32768
BC = 1024
NB = C // BC
INT_MAX = 2147483647


def _body(x_ref, out_ref, m_scr, i_scr, zsc, fix_scr, ismem, zsem, isem, fsem):
    j = pl.program_id(0)
    blk = x_ref[...]
    m = jnp.max(blk, axis=1, keepdims=True)
    liota = lax.broadcasted_iota(jnp.int32, blk.shape, 1) + j * BC
    cand = jnp.where(blk == m, liota, INT_MAX)
    ci = jnp.min(cand, axis=1, keepdims=True)

    @pl.when(j == 0)
    def _():
        m_scr[...] = m
        i_scr[...] = ci
        zsc[...] = jnp.zeros((R, BC), jnp.float32)

    @pl.when(j > 0)
    def _():
        upd = m > m_scr[...]
        i_scr[...] = jnp.where(upd, ci, i_scr[...])
        m_scr[...] = jnp.where(upd, m, m_scr[...])

    pltpu.make_async_copy(zsc, out_ref.at[:, pl.ds(j * BC, BC)], zsem).start()

    @pl.when(j == NB - 1)
    def _():
        pltpu.make_async_copy(i_scr, ismem, isem).start()
        lane = lax.broadcasted_iota(jnp.int32, (R, 128), 1)
        fix_scr[...] = jnp.where(
            lane == i_scr[...] % 128, 1.0, 0.0
        ).astype(jnp.float32)

        def zdrain(_, c):
            pltpu.make_async_copy(
                zsc, out_ref.at[:, pl.ds(0, BC)], zsem
            ).wait()
            return c

        lax.fori_loop(0, NB, zdrain, 0)
        pltpu.make_async_copy(i_scr, ismem, isem).wait()

        def fire(r, c):
            base = (ismem[r, 0] // 128) * 128
            pltpu.make_async_copy(
                fix_scr.at[pl.ds(r, 1), :],
                out_ref.at[pl.ds(r, 1), pl.ds(base, 128)],
                fsem,
            ).start()
            return c

        lax.fori_loop(0, R, fire, 0)

        def fdrain(_, c):
            pltpu.make_async_copy(
                fix_scr.at[pl.ds(0, 1), :],
                out_ref.at[pl.ds(0, 1), pl.ds(0, 128)],
                fsem,
            ).wait()
            return c

        lax.fori_loop(0, R, fdrain, 0)


def kernel(x):
    return pl.pallas_call(
        _body,
        grid=(NB,),
        in_specs=[pl.BlockSpec((R, BC), lambda j: (0, j))],
        out_specs=pl.BlockSpec(memory_space=pl.ANY),
        out_shape=jax.ShapeDtypeStruct((R, C), jnp.float32),
        scratch_shapes=[
            pltpu.VMEM((R, 1), jnp.float32),
            pltpu.VMEM((R, 1), jnp.int32),
            pltpu.VMEM((R, BC), jnp.float32),
            pltpu.VMEM((R, 128), jnp.float32),
            pltpu.SMEM((R, 1), jnp.int32),
            pltpu.SemaphoreType.DMA,
            pltpu.SemaphoreType.DMA,
            pltpu.SemaphoreType.DMA,
        ],
        compiler_params=pltpu.CompilerParams(
            dimension_semantics=("arbitrary",),
        ),
    )(x)


# speedup vs baseline: 4.0832x; 1.8569x over previous
import jax
import jax.numpy as jnp
from jax import lax
from jax.experimental import pallas as pl
from jax.experimental.pallas import tpu as pltpu

R, C = 128, 32768
BC = 4096
NB = C // BC
INT_MAX = 2147483647


def _body(x_ref, out_ref, m_scr, i_scr, zsc, fix_scr, ismem, zsem, isem, fsem):
    j = pl.program_id(0)
    blk = x_ref[...]
    m = jnp.max(blk, axis=1, keepdims=True)
    liota = lax.broadcasted_iota(jnp.int32, blk.shape, 1)
    cand = jnp.where(blk == m, liota, INT_MAX)
    ci = jnp.min(cand, axis=1, keepdims=True) + j * BC

    @pl.when(j == 0)
    def _():
        m_scr[...] = m
        i_scr[...] = ci
        zsc[...] = jnp.zeros((R, BC), jnp.float32)

    @pl.when(j > 0)
    def _():
        upd = m > m_scr[...]
        i_scr[...] = jnp.where(upd, ci, i_scr[...])
        m_scr[...] = jnp.where(upd, m, m_scr[...])

    pltpu.make_async_copy(zsc, out_ref.at[:, pl.ds(j * BC, BC)], zsem).start()

    @pl.when(j == NB - 1)
    def _():
        pltpu.make_async_copy(i_scr, ismem, isem).start()
        lane = lax.broadcasted_iota(jnp.int32, (R, 128), 1)
        fix_scr[...] = jnp.where(
            lane == i_scr[...] % 128, 1.0, 0.0
        ).astype(jnp.float32)

        def zdrain(_, c):
            pltpu.make_async_copy(
                zsc, out_ref.at[:, pl.ds(0, BC)], zsem
            ).wait()
            return c

        lax.fori_loop(0, NB, zdrain, 0)
        pltpu.make_async_copy(i_scr, ismem, isem).wait()

        def fire(r, c):
            base = (ismem[r, 0] // 128) * 128
            pltpu.make_async_copy(
                fix_scr.at[pl.ds(r, 1), :],
                out_ref.at[pl.ds(r, 1), pl.ds(base, 128)],
                fsem,
            ).start()
            return c

        lax.fori_loop(0, R, fire, 0)

        def fdrain(_, c):
            pltpu.make_async_copy(
                fix_scr.at[pl.ds(0, 1), :],
                out_ref.at[pl.ds(0, 1), pl.ds(0, 128)],
                fsem,
            ).wait()
            return c

        lax.fori_loop(0, R, fdrain, 0)


def kernel(x):
    return pl.pallas_call(
        _body,
        grid=(NB,),
        in_specs=[pl.BlockSpec((R, BC), lambda j: (0, j))],
        out_specs=pl.BlockSpec(memory_space=pl.ANY),
        out_shape=jax.ShapeDtypeStruct((R, C), jnp.float32),
        scratch_shapes=[
            pltpu.VMEM((R, 1), jnp.float32),
            pltpu.VMEM((R, 1), jnp.int32),
            pltpu.VMEM((R, BC), jnp.float32),
            pltpu.VMEM((R, 128), jnp.float32),
            pltpu.SMEM((R, 1), jnp.int32),
            pltpu.SemaphoreType.DMA,
            pltpu.SemaphoreType.DMA,
            pltpu.SemaphoreType.DMA,
        ],
        compiler_params=pltpu.CompilerParams(
            dimension_semantics=("arbitrary",),
        ),
    )(x)


# speedup vs baseline: 4.4244x; 1.0836x over previous
import jax
import jax.numpy as jnp
from jax import lax
from jax.experimental import pallas as pl
from jax.experimental.pallas import tpu as pltpu

R, C = 128, 32768
BC = 8192
NB = C // BC
INT_MAX = 2147483647


def _body(x_ref, out_ref, m_scr, i_scr, zsc, fix_scr, ismem, zsem, isem, fsem):
    j = pl.program_id(0)
    blk = x_ref[...]
    m = jnp.max(blk, axis=1, keepdims=True)
    liota = lax.broadcasted_iota(jnp.int32, blk.shape, 1)
    cand = jnp.where(blk == m, liota, INT_MAX)
    ci = jnp.min(cand, axis=1, keepdims=True) + j * BC

    @pl.when(j == 0)
    def _():
        m_scr[...] = m
        i_scr[...] = ci
        zsc[...] = jnp.zeros((R, BC), jnp.float32)

    @pl.when(j > 0)
    def _():
        upd = m > m_scr[...]
        i_scr[...] = jnp.where(upd, ci, i_scr[...])
        m_scr[...] = jnp.where(upd, m, m_scr[...])

    pltpu.make_async_copy(zsc, out_ref.at[:, pl.ds(j * BC, BC)], zsem).start()

    @pl.when(j == NB - 1)
    def _():
        pltpu.make_async_copy(i_scr, ismem, isem).start()
        lane = lax.broadcasted_iota(jnp.int32, (R, 128), 1)
        fix_scr[...] = jnp.where(
            lane == i_scr[...] % 128, 1.0, 0.0
        ).astype(jnp.float32)

        def zdrain(_, c):
            pltpu.make_async_copy(
                zsc, out_ref.at[:, pl.ds(0, BC)], zsem
            ).wait()
            return c

        lax.fori_loop(0, NB, zdrain, 0)
        pltpu.make_async_copy(i_scr, ismem, isem).wait()

        def fire(r, c):
            base = (ismem[r, 0] // 128) * 128
            pltpu.make_async_copy(
                fix_scr.at[pl.ds(r, 1), :],
                out_ref.at[pl.ds(r, 1), pl.ds(base, 128)],
                fsem,
            ).start()
            return c

        lax.fori_loop(0, R, fire, 0)

        def fdrain(_, c):
            pltpu.make_async_copy(
                fix_scr.at[pl.ds(0, 1), :],
                out_ref.at[pl.ds(0, 1), pl.ds(0, 128)],
                fsem,
            ).wait()
            return c

        lax.fori_loop(0, R, fdrain, 0)


def kernel(x):
    return pl.pallas_call(
        _body,
        grid=(NB,),
        in_specs=[pl.BlockSpec((R, BC), lambda j: (0, j))],
        out_specs=pl.BlockSpec(memory_space=pl.ANY),
        out_shape=jax.ShapeDtypeStruct((R, C), jnp.float32),
        scratch_shapes=[
            pltpu.VMEM((R, 1), jnp.float32),
            pltpu.VMEM((R, 1), jnp.int32),
            pltpu.VMEM((R, BC), jnp.float32),
            pltpu.VMEM((R, 128), jnp.float32),
            pltpu.SMEM((R, 1), jnp.int32),
            pltpu.SemaphoreType.DMA,
            pltpu.SemaphoreType.DMA,
            pltpu.SemaphoreType.DMA,
        ],
        compiler_params=pltpu.CompilerParams(
            dimension_semantics=("arbitrary",),
        ),
    )(x)


# speedup vs baseline: 4.8965x; 1.1067x over previous
import jax
import jax.numpy as jnp
from jax import lax
from jax.experimental import pallas as pl
from jax.experimental.pallas import tpu as pltpu

R, C = 128, 32768
BC = 16384
NB = C // BC
INT_MAX = 2147483647


def _body(x_ref, out_ref, m_scr, i_scr, zsc, fix_scr, ismem, zsem, isem, fsem):
    j = pl.program_id(0)
    blk = x_ref[...]
    m = jnp.max(blk, axis=1, keepdims=True)
    liota = lax.broadcasted_iota(jnp.int32, blk.shape, 1)
    cand = jnp.where(blk == m, liota, INT_MAX)
    ci = jnp.min(cand, axis=1, keepdims=True) + j * BC

    @pl.when(j == 0)
    def _():
        m_scr[...] = m
        i_scr[...] = ci
        zsc[...] = jnp.zeros((R, BC), jnp.float32)

    @pl.when(j > 0)
    def _():
        upd = m > m_scr[...]
        i_scr[...] = jnp.where(upd, ci, i_scr[...])
        m_scr[...] = jnp.where(upd, m, m_scr[...])

    pltpu.make_async_copy(zsc, out_ref.at[:, pl.ds(j * BC, BC)], zsem).start()

    @pl.when(j == NB - 1)
    def _():
        pltpu.make_async_copy(i_scr, ismem, isem).start()
        lane = lax.broadcasted_iota(jnp.int32, (R, 128), 1)
        fix_scr[...] = jnp.where(
            lane == i_scr[...] % 128, 1.0, 0.0
        ).astype(jnp.float32)

        def zdrain(_, c):
            pltpu.make_async_copy(
                zsc, out_ref.at[:, pl.ds(0, BC)], zsem
            ).wait()
            return c

        lax.fori_loop(0, NB, zdrain, 0)
        pltpu.make_async_copy(i_scr, ismem, isem).wait()

        def fire(r, c):
            base = (ismem[r, 0] // 128) * 128
            pltpu.make_async_copy(
                fix_scr.at[pl.ds(r, 1), :],
                out_ref.at[pl.ds(r, 1), pl.ds(base, 128)],
                fsem,
            ).start()
            return c

        lax.fori_loop(0, R, fire, 0)

        def fdrain(_, c):
            pltpu.make_async_copy(
                fix_scr.at[pl.ds(0, 1), :],
                out_ref.at[pl.ds(0, 1), pl.ds(0, 128)],
                fsem,
            ).wait()
            return c

        lax.fori_loop(0, R, fdrain, 0)


def kernel(x):
    return pl.pallas_call(
        _body,
        grid=(NB,),
        in_specs=[pl.BlockSpec((R, BC), lambda j: (0, j))],
        out_specs=pl.BlockSpec(memory_space=pl.ANY),
        out_shape=jax.ShapeDtypeStruct((R, C), jnp.float32),
        scratch_shapes=[
            pltpu.VMEM((R, 1), jnp.float32),
            pltpu.VMEM((R, 1), jnp.int32),
            pltpu.VMEM((R, BC), jnp.float32),
            pltpu.VMEM((R, 128), jnp.float32),
            pltpu.SMEM((R, 1), jnp.int32),
            pltpu.SemaphoreType.DMA,
            pltpu.SemaphoreType.DMA,
            pltpu.SemaphoreType.DMA,
        ],
        compiler_params=pltpu.CompilerParams(
            dimension_semantics=("arbitrary",),
        ),
    )(x)
